# baseline (device time: 36449 ns/iter reference)
import functools

import jax
import jax.numpy as jnp
from jax import lax
from jax.experimental import pallas as pl
from jax.experimental.pallas import tpu as pltpu

N_DEV = 8
N_ROUNDS = 3
N_LAYERS = 3
B = 64
D = 1024
H = 2048
MASKS = (1, 3, 4)


def kernel(x, Win0, Wout0, Win1, Wout1, Win2, Wout2):
    def body(
        x_ref,
        win0_ref,
        wout0_ref,
        win1_ref,
        wout1_ref,
        win2_ref,
        wout2_ref,
        out_ref,
        win_stage,
        wout_stage,
        acc_ref,
        send_buf,
        recv_buf,
        rs_send,
        rs_recv,
        head_sems,
        load_sems,
        send_sems,
        recv_sems,
        rs_send_sems,
        rs_recv_sems,
    ):
        my = lax.axis_index("i")

        wins = [win0_ref, win1_ref, win2_ref]
        wouts = [wout0_ref, wout1_ref, wout2_ref]

        def stage(l, slot):
            cp_in = pltpu.make_async_copy(
                wins[l], win_stage.at[slot], load_sems.at[slot, 0]
            )
            cp_out = pltpu.make_async_copy(
                wouts[l], wout_stage.at[slot], load_sems.at[slot, 1]
            )
            cp_in.start()
            cp_out.start()
            return cp_in, cp_out

        HH = H // 2
        head_cps = [
            pltpu.make_async_copy(
                win0_ref.at[:, pl.ds(0, HH)],
                win_stage.at[0, :, pl.ds(0, HH)],
                head_sems.at[0],
            ),
            pltpu.make_async_copy(
                win0_ref.at[:, pl.ds(HH, HH)],
                win_stage.at[0, :, pl.ds(HH, HH)],
                head_sems.at[1],
            ),
            pltpu.make_async_copy(
                wout0_ref.at[pl.ds(0, HH), :],
                wout_stage.at[0, pl.ds(0, HH), :],
                head_sems.at[2],
            ),
            pltpu.make_async_copy(
                wout0_ref.at[pl.ds(HH, HH), :],
                wout_stage.at[0, pl.ds(HH, HH), :],
                head_sems.at[3],
            ),
        ]
        for cp in head_cps:
            cp.start()

        barrier = pltpu.get_barrier_semaphore()
        for o in range(1, N_DEV):
            pl.semaphore_signal(
                barrier,
                inc=1,
                device_id=(my ^ o,),
                device_id_type=pl.DeviceIdType.MESH,
            )
        pl.semaphore_wait(barrier, N_DEV - 1)

        HALF = D // 2

        def half_exchange(l, r, half, partner, val):
            idx = (l * N_ROUNDS + r) * 2 + half
            send_buf[idx, :, :] = val.astype(jnp.bfloat16)
            rdma = pltpu.make_async_remote_copy(
                src_ref=send_buf.at[idx],
                dst_ref=recv_buf.at[idx],
                send_sem=send_sems.at[idx],
                recv_sem=recv_sems.at[idx],
                device_id=(partner,),
                device_id_type=pl.DeviceIdType.MESH,
            )
            rdma.start()
            return rdma, idx

        x_bf = x_ref[:, :].astype(jnp.bfloat16)
        acc = None
        for l in range(N_LAYERS):
            slot = l % 2
            if l == 0:
                head_cps[0].wait()
                h1 = jnp.maximum(
                    jnp.dot(
                        x_bf,
                        win_stage[0, :, :HH].astype(jnp.bfloat16),
                        preferred_element_type=jnp.float32,
                    ),
                    0.0,
                ).astype(jnp.bfloat16)
                head_cps[1].wait()
                h2 = jnp.maximum(
                    jnp.dot(
                        x_bf,
                        win_stage[0, :, HH:].astype(jnp.bfloat16),
                        preferred_element_type=jnp.float32,
                    ),
                    0.0,
                ).astype(jnp.bfloat16)
                head_cps[2].wait()
                acc = jnp.dot(
                    h1,
                    wout_stage[0, :HH, :].astype(jnp.bfloat16),
                    preferred_element_type=jnp.float32,
                )
                head_cps[3].wait()
                acc = acc + jnp.dot(
                    h2,
                    wout_stage[0, HH:, :].astype(jnp.bfloat16),
                    preferred_element_type=jnp.float32,
                )
            else:
                h = jnp.maximum(
                    jnp.dot(x_bf, win_bf, preferred_element_type=jnp.float32),
                    0.0,
                ).astype(jnp.bfloat16)
                acc = jnp.dot(h, wout_bf, preferred_element_type=jnp.float32)
            if l + 1 < N_LAYERS:
                cp_in, cp_out = stage(l + 1, 1 - slot)
                ha = acc[:, :HALF]
                hb = acc[:, HALF:]
                rd_a, ia = half_exchange(l, 0, 0, my ^ MASKS[0], ha)
                rd_b, ib = half_exchange(l, 0, 1, my ^ MASKS[0], hb)
                for r in range(N_ROUNDS):
                    if r == 1:
                        cp_in.wait()
                        win_bf = win_stage[1 - slot, :, :].astype(jnp.bfloat16)
                    elif r == 2:
                        cp_out.wait()
                        wout_bf = wout_stage[1 - slot, :, :].astype(
                            jnp.bfloat16
                        )
                    rd_a.wait()
                    ha = ha + recv_buf[ia, :, :].astype(jnp.float32)
                    if r + 1 < N_ROUNDS:
                        rd_a, ia = half_exchange(
                            l, r + 1, 0, my ^ MASKS[r + 1], ha
                        )
                    rd_b.wait()
                    hb = hb + recv_buf[ib, :, :].astype(jnp.float32)
                    if r + 1 < N_ROUNDS:
                        rd_b, ib = half_exchange(
                            l, r + 1, 1, my ^ MASKS[r + 1], hb
                        )
                acc = jnp.concatenate([ha, hb], axis=1)
                x_bf = acc.astype(jnp.bfloat16)

        acc_ref[:, :] = acc

        rows = B // N_DEV
        rdmas = []
        for o in range(1, N_DEV):
            q = my ^ o
            rs_send[o, :, :] = acc_ref[pl.ds(q * rows, rows), :].astype(
                jnp.bfloat16
            )
            rdma = pltpu.make_async_remote_copy(
                src_ref=rs_send.at[o],
                dst_ref=rs_recv.at[o],
                send_sem=rs_send_sems.at[o],
                recv_sem=rs_recv_sems.at[o],
                device_id=(q,),
                device_id_type=pl.DeviceIdType.MESH,
            )
            rdma.start()
            rdmas.append(rdma)
        mine = acc_ref[pl.ds(my * rows, rows), :]
        for o, rdma in enumerate(rdmas, start=1):
            rdma.wait()
            mine = mine + rs_recv[o, :, :].astype(jnp.float32)
        out_ref[:, :] = mine

        @functools.partial(pl.run_scoped, exit_sem=pltpu.SemaphoreType.REGULAR)
        def _(exit_sem):
            for o in range(1, N_DEV):
                pl.semaphore_signal(
                    exit_sem,
                    inc=1,
                    device_id=(my ^ o,),
                    device_id_type=pl.DeviceIdType.MESH,
                )
            pl.semaphore_wait(exit_sem, N_DEV - 1)

    hbm = pl.BlockSpec(memory_space=pltpu.MemorySpace.HBM)
    vmem = pl.BlockSpec(memory_space=pltpu.VMEM)
    n_ex = N_LAYERS * N_ROUNDS
    return pl.pallas_call(
        body,
        out_shape=jax.ShapeDtypeStruct((B // N_DEV, D), jnp.float32),
        in_specs=[vmem, hbm, hbm, hbm, hbm, hbm, hbm],
        out_specs=vmem,
        scratch_shapes=[
            pltpu.VMEM((2, D, H), jnp.float32),
            pltpu.VMEM((2, H, D), jnp.float32),
            pltpu.VMEM((B, D), jnp.float32),
            pltpu.VMEM((12, B, D // 2), jnp.bfloat16),
            pltpu.VMEM((12, B, D // 2), jnp.bfloat16),
            pltpu.VMEM((N_DEV, B // N_DEV, D), jnp.bfloat16),
            pltpu.VMEM((N_DEV, B // N_DEV, D), jnp.bfloat16),
            pltpu.SemaphoreType.DMA((4,)),
            pltpu.SemaphoreType.DMA((2, 2)),
            pltpu.SemaphoreType.DMA((12,)),
            pltpu.SemaphoreType.DMA((12,)),
            pltpu.SemaphoreType.DMA((N_DEV,)),
            pltpu.SemaphoreType.DMA((N_DEV,)),
        ],
        compiler_params=pltpu.CompilerParams(
            collective_id=0, vmem_limit_bytes=56 * 1024 * 1024
        ),
    )(x, Win0, Wout0, Win1, Wout1, Win2, Wout2)


# device time: 36136 ns/iter; 1.0087x vs baseline; 1.0087x over previous
import functools

import jax
import jax.numpy as jnp
from jax import lax
from jax.experimental import pallas as pl
from jax.experimental.pallas import tpu as pltpu

N_DEV = 8
N_ROUNDS = 3
N_LAYERS = 3
B = 64
D = 1024
H = 2048
MASKS = (1, 3, 4)


def kernel(x, Win0, Wout0, Win1, Wout1, Win2, Wout2):
    def body(
        x_ref,
        win0_ref,
        wout0_ref,
        win1_ref,
        wout1_ref,
        win2_ref,
        wout2_ref,
        out_ref,
        win_stage,
        wout_stage,
        acc_ref,
        send_buf,
        recv_buf,
        rs_send,
        rs_recv,
        head_sems,
        load_sems,
        send_sems,
        recv_sems,
        rs_send_sems,
        rs_recv_sems,
    ):
        my = lax.axis_index("i")

        wins = [win0_ref, win1_ref, win2_ref]
        wouts = [wout0_ref, wout1_ref, wout2_ref]

        def stage(l, slot):
            cp_in = pltpu.make_async_copy(
                wins[l], win_stage.at[slot], load_sems.at[slot, 0]
            )
            cp_out = pltpu.make_async_copy(
                wouts[l], wout_stage.at[slot], load_sems.at[slot, 1]
            )
            cp_in.start()
            cp_out.start()
            return cp_in, cp_out

        HH = H // 2
        head_cps = [
            pltpu.make_async_copy(
                win0_ref.at[:, pl.ds(0, HH)],
                win_stage.at[0, :, pl.ds(0, HH)],
                head_sems.at[0],
            ),
            pltpu.make_async_copy(
                win0_ref.at[:, pl.ds(HH, HH)],
                win_stage.at[0, :, pl.ds(HH, HH)],
                head_sems.at[1],
            ),
            pltpu.make_async_copy(
                wout0_ref.at[pl.ds(0, HH), :],
                wout_stage.at[0, pl.ds(0, HH), :],
                head_sems.at[2],
            ),
            pltpu.make_async_copy(
                wout0_ref.at[pl.ds(HH, HH), :],
                wout_stage.at[0, pl.ds(HH, HH), :],
                head_sems.at[3],
            ),
        ]
        for cp in head_cps:
            cp.start()

        barrier = pltpu.get_barrier_semaphore()
        for o in range(1, N_DEV):
            pl.semaphore_signal(
                barrier,
                inc=1,
                device_id=(my ^ o,),
                device_id_type=pl.DeviceIdType.MESH,
            )
        pl.semaphore_wait(barrier, N_DEV - 1)

        HALF = D // 2

        def half_exchange(l, r, half, partner, val):
            idx = (l * N_ROUNDS + r) * 2 + half
            send_buf[idx, :, :] = val.astype(jnp.bfloat16)
            rdma = pltpu.make_async_remote_copy(
                src_ref=send_buf.at[idx],
                dst_ref=recv_buf.at[idx],
                send_sem=send_sems.at[idx],
                recv_sem=recv_sems.at[idx],
                device_id=(partner,),
                device_id_type=pl.DeviceIdType.MESH,
            )
            rdma.start()
            return rdma, idx

        x_bf = x_ref[:, :].astype(jnp.bfloat16)
        acc = None
        for l in range(N_LAYERS):
            slot = l % 2
            if l == 0:
                head_cps[0].wait()
                h1 = jnp.maximum(
                    jnp.dot(
                        x_bf,
                        win_stage[0, :, :HH].astype(jnp.bfloat16),
                        preferred_element_type=jnp.float32,
                    ),
                    0.0,
                ).astype(jnp.bfloat16)
                head_cps[1].wait()
                h2 = jnp.maximum(
                    jnp.dot(
                        x_bf,
                        win_stage[0, :, HH:].astype(jnp.bfloat16),
                        preferred_element_type=jnp.float32,
                    ),
                    0.0,
                ).astype(jnp.bfloat16)
                head_cps[2].wait()
                acc = jnp.dot(
                    h1,
                    wout_stage[0, :HH, :].astype(jnp.bfloat16),
                    preferred_element_type=jnp.float32,
                )
                head_cps[3].wait()
                acc = acc + jnp.dot(
                    h2,
                    wout_stage[0, HH:, :].astype(jnp.bfloat16),
                    preferred_element_type=jnp.float32,
                )
            else:
                h = jnp.maximum(
                    jnp.dot(x_bf, win_bf, preferred_element_type=jnp.float32),
                    0.0,
                ).astype(jnp.bfloat16)
                acc = jnp.dot(h, wout_bf, preferred_element_type=jnp.float32)
            if l + 1 < N_LAYERS:
                cp_in, cp_out = stage(l + 1, 1 - slot)
                ha = acc[:, :HALF]
                hb = acc[:, HALF:]
                rd_a, ia = half_exchange(l, 0, 0, my ^ MASKS[0], ha)
                rd_b, ib = half_exchange(l, 0, 1, my ^ MASKS[0], hb)
                for r in range(N_ROUNDS):
                    if r == 1:
                        cp_in.wait()
                        win_bf = win_stage[1 - slot, :, :].astype(jnp.bfloat16)
                    elif r == 2:
                        cp_out.wait()
                        wout_bf = wout_stage[1 - slot, :, :].astype(
                            jnp.bfloat16
                        )
                    rd_a.wait()
                    ha = ha + recv_buf[ia, :, :].astype(jnp.float32)
                    if r + 1 < N_ROUNDS:
                        rd_a, ia = half_exchange(
                            l, r + 1, 0, my ^ MASKS[r + 1], ha
                        )
                    rd_b.wait()
                    hb = hb + recv_buf[ib, :, :].astype(jnp.float32)
                    if r + 1 < N_ROUNDS:
                        rd_b, ib = half_exchange(
                            l, r + 1, 1, my ^ MASKS[r + 1], hb
                        )
                acc = jnp.concatenate([ha, hb], axis=1)
                x_bf = acc.astype(jnp.bfloat16)

        acc_ref[:, :] = acc

        rows = B // N_DEV
        rdmas = {}
        for o in (6, 5, 7, 2, 1, 3, 4):
            q = my ^ o
            rs_send[o, :, :] = acc_ref[pl.ds(q * rows, rows), :].astype(
                jnp.bfloat16
            )
            rdma = pltpu.make_async_remote_copy(
                src_ref=rs_send.at[o],
                dst_ref=rs_recv.at[o],
                send_sem=rs_send_sems.at[o],
                recv_sem=rs_recv_sems.at[o],
                device_id=(q,),
                device_id_type=pl.DeviceIdType.MESH,
            )
            rdma.start()
            rdmas[o] = rdma
        mine = acc_ref[pl.ds(my * rows, rows), :]
        for o in (1, 3, 4, 2, 5, 7, 6):
            rdmas[o].wait()
            mine = mine + rs_recv[o, :, :].astype(jnp.float32)
        out_ref[:, :] = mine

        @functools.partial(pl.run_scoped, exit_sem=pltpu.SemaphoreType.REGULAR)
        def _(exit_sem):
            for o in range(1, N_DEV):
                pl.semaphore_signal(
                    exit_sem,
                    inc=1,
                    device_id=(my ^ o,),
                    device_id_type=pl.DeviceIdType.MESH,
                )
            pl.semaphore_wait(exit_sem, N_DEV - 1)

    hbm = pl.BlockSpec(memory_space=pltpu.MemorySpace.HBM)
    vmem = pl.BlockSpec(memory_space=pltpu.VMEM)
    n_ex = N_LAYERS * N_ROUNDS
    return pl.pallas_call(
        body,
        out_shape=jax.ShapeDtypeStruct((B // N_DEV, D), jnp.float32),
        in_specs=[vmem, hbm, hbm, hbm, hbm, hbm, hbm],
        out_specs=vmem,
        scratch_shapes=[
            pltpu.VMEM((2, D, H), jnp.float32),
            pltpu.VMEM((2, H, D), jnp.float32),
            pltpu.VMEM((B, D), jnp.float32),
            pltpu.VMEM((12, B, D // 2), jnp.bfloat16),
            pltpu.VMEM((12, B, D // 2), jnp.bfloat16),
            pltpu.VMEM((N_DEV, B // N_DEV, D), jnp.bfloat16),
            pltpu.VMEM((N_DEV, B // N_DEV, D), jnp.bfloat16),
            pltpu.SemaphoreType.DMA((4,)),
            pltpu.SemaphoreType.DMA((2, 2)),
            pltpu.SemaphoreType.DMA((12,)),
            pltpu.SemaphoreType.DMA((12,)),
            pltpu.SemaphoreType.DMA((N_DEV,)),
            pltpu.SemaphoreType.DMA((N_DEV,)),
        ],
        compiler_params=pltpu.CompilerParams(
            collective_id=0, vmem_limit_bytes=56 * 1024 * 1024
        ),
    )(x, Win0, Wout0, Win1, Wout1, Win2, Wout2)
